# baseline (speedup 1.0000x reference)
import functools
import jax
import jax.numpy as jnp
from jax import lax
from jax.experimental import pallas as pl
from jax.experimental.pallas import tpu as pltpu
from jax.experimental.pallas import tpu_sc as plsc

M = 9
F = 128
CN = 16
LANES = 16
NW = 32


def kernel(X1, X2, clebsch):
    m, n, f = X1.shape
    n_chunks = n // CN
    pairs = [(m1, k - m1) for k in range(M) for m1 in range(k + 1)]
    c_rows = jnp.stack([jnp.full((LANES,), clebsch[m1, m2]) for (m1, m2) in pairs])

    mesh = plsc.VectorSubcoreMesh(
        core_axis_name="c", subcore_axis_name="s", num_cores=2, num_subcores=16
    )

    @functools.partial(
        pl.kernel,
        mesh=mesh,
        out_type=jax.ShapeDtypeStruct((M, n, f), jnp.float32),
        scratch_types=[
            pltpu.VMEM((M, CN, F), jnp.float32),
            pltpu.VMEM((M, CN, F), jnp.float32),
            pltpu.VMEM((M, CN, F), jnp.float32),
            pltpu.VMEM((len(pairs), LANES), jnp.float32),
            pltpu.SemaphoreType.DMA,
        ],
    )
    def body(x1_hbm, x2_hbm, c_hbm, out_hbm, x1_v, x2_v, o_v, c_v, sem):
        wid = lax.axis_index("s") * 2 + lax.axis_index("c")
        pltpu.sync_copy(c_hbm, c_v)

        def do_chunk(ci, _):
            base = ci * CN
            descs = []
            for mm in range(M):
                descs.append(
                    pltpu.async_copy(x1_hbm.at[mm, pl.ds(base, CN)], x1_v.at[mm], sem)
                )
                descs.append(
                    pltpu.async_copy(x2_hbm.at[mm, pl.ds(base, CN)], x2_v.at[mm], sem)
                )
            for dsc in descs:
                dsc.wait()

            def do_vec(p, _):
                r = p // 8
                j = (p % 8) * LANES
                x1r = [x1_v[mm, r, pl.ds(j, LANES)] for mm in range(M)]
                x2r = [x2_v[mm, r, pl.ds(j, LANES)] for mm in range(M)]
                accs = [None] * M
                pi = 0
                for k in range(M):
                    for m1 in range(k + 1):
                        m2 = k - m1
                        t = x1r[m1] * x2r[m2] * c_v[pi]
                        accs[k] = t if accs[k] is None else accs[k] + t
                        pi += 1
                for k in range(M):
                    o_v[k, r, pl.ds(j, LANES)] = accs[k]
                return 0

            lax.fori_loop(0, CN * 8, do_vec, 0)
            odescs = [
                pltpu.async_copy(o_v.at[mm], out_hbm.at[mm, pl.ds(base, CN)], sem)
                for mm in range(M)
            ]
            for dsc in odescs:
                dsc.wait()
            return 0

        nc_w = (n_chunks - wid + NW - 1) // NW
        lax.fori_loop(0, nc_w, lambda i, _: do_chunk(wid + i * NW, _), 0)

    return body(X1, X2, c_rows)
